# baseline (speedup 1.0000x reference)
import jax
import jax.numpy as jnp
from jax import lax
from jax.experimental import pallas as pl
from jax.experimental.pallas import tpu as pltpu
from jax.experimental.pallas import tpu_sc as plsc

_OLD_VOCAB = 1000000
_NEW_VOCAB = 1000
_EMBED_DIM = 64

_NUM_WORKERS = 32
_STEP = 256
_SUB = 128
_LANES = 16


def _body(ids_hbm, old_hbm, new_hbm, out_hbm,
          newtab_v, idbuf_v, idxbuf_v, buf_v, bufT_v, isems, gsems, wsem):
    n_seq = ids_hbm.shape[0]
    batch = ids_hbm.shape[1]
    per_w = batch // _NUM_WORKERS
    halves = per_w // _STEP
    n_steps = n_seq * halves
    wid = lax.axis_index("s") * 2 + lax.axis_index("c")
    wb = wid * per_w

    pltpu.sync_copy(new_hbm, newtab_v)

    iota16 = lax.iota(jnp.int32, _LANES)
    dvecs = [iota16 + dd * _LANES for dd in range(_EMBED_DIM // _LANES)]

    def ids_slice(t):
        s = t // halves
        col = wb + (t % halves) * _STEP
        return ids_hbm.at[s, pl.ds(col, _STEP)]

    def idbuf(b):
        return idbuf_v.at[pl.ds(b * _STEP, _STEP)]

    def fire_ids(t, b):
        pltpu.async_copy(ids_slice(t), idbuf(b), isems[b])

    def wait_ids(t, b):
        pltpu.make_async_copy(ids_slice(t), idbuf(b), isems[b]).wait()

    def clip(b):
        def clip_body(i, _):
            v = idbuf_v[pl.ds(b * _STEP + i * _LANES, _LANES)]
            idxbuf_v[pl.ds(b * _STEP + i * _LANES, _LANES)] = (
                jnp.minimum(v, _OLD_VOCAB - 1))
            return 0
        lax.fori_loop(0, _STEP // _LANES, clip_body, 0)

    def gather_pairs(b):
        out = []
        for j in range(_STEP // _SUB):
            out.append((
                old_hbm.at[idxbuf_v.at[pl.ds(b * _STEP + j * _SUB, _SUB)]],
                buf_v.at[pl.ds(b * _STEP + j * _SUB, _SUB)]))
        return out

    def fire_gathers(b):
        for src, dst in gather_pairs(b):
            pltpu.async_copy(src, dst, gsems[b])

    def wait_gathers(b):
        for src, dst in gather_pairs(b):
            pltpu.make_async_copy(src, dst, gsems[b]).wait()

    def fixup(b):
        def fix_body(i, _):
            v = idbuf_v[pl.ds(b * _STEP + i * _LANES, _LANES)]
            m = v >= _OLD_VOCAB
            cnt = plsc.all_reduce_population_count(m)

            @pl.when(cnt[0] > 0)
            def _():
                nid = jnp.maximum(v - _OLD_VOCAB, 0)
                rowpos = iota16 + (b * _STEP + i * _LANES)

                def d_body(d, _):
                    dvec = jnp.full((_LANES,), d, jnp.int32)
                    vals = plsc.load_gather(newtab_v, [nid, dvec], mask=m)
                    plsc.store_scatter(buf_v, [rowpos, dvec], vals, mask=m)
                    return 0
                lax.fori_loop(0, _EMBED_DIM, d_body, 0)
            return 0
        lax.fori_loop(0, _STEP // _LANES, fix_body, 0)

    def transpose(b):
        base = b * _STEP

        def t_body(bb, _):
            for u in range(2):
                row = bb * 2 + u
                bvec = jnp.full((_LANES,), row, jnp.int32)
                for dd in range(_EMBED_DIM // _LANES):
                    vals = buf_v[base + row, pl.ds(dd * _LANES, _LANES)]
                    plsc.store_scatter(bufT_v, [dvecs[dd], bvec], vals)
            return 0
        lax.fori_loop(0, _STEP // 2, t_body, 0)

    def out_slice(t):
        s = t // halves
        col = wb + (t % halves) * _STEP
        return out_hbm.at[s, :, pl.ds(col, _STEP)]

    fire_ids(0, 0)
    fire_ids(1, 1)
    wait_ids(0, 0)
    clip(0)
    fire_gathers(0)

    def step(tp, _):
        for b in range(2):
            t = tp * 2 + b
            nb = 1 - b

            @pl.when(t + 1 < n_steps)
            def _():
                wait_ids(t + 1, nb)
                clip(nb)
                fire_gathers(nb)

            wait_gathers(b)
            fixup(b)

            @pl.when(t >= 1)
            def _():
                pltpu.make_async_copy(bufT_v, out_slice(t - 1), wsem).wait()

            transpose(b)
            pltpu.async_copy(bufT_v, out_slice(t), wsem)

            @pl.when(t + 2 < n_steps)
            def _():
                fire_ids(t + 2, b)
        return 0

    lax.fori_loop(0, n_steps // 2, step, 0)
    pltpu.make_async_copy(bufT_v, out_slice(n_steps - 1), wsem).wait()


def kernel(input_ids, old_table, new_table):
    batch, seq = input_ids.shape
    ids_t = input_ids.T

    mesh = plsc.VectorSubcoreMesh(core_axis_name="c", subcore_axis_name="s")
    run = pl.kernel(
        _body,
        out_type=jax.ShapeDtypeStruct((seq, _EMBED_DIM, batch), jnp.float32),
        mesh=mesh,
        compiler_params=pltpu.CompilerParams(
            needs_layout_passes=False, use_tc_tiling_on_sc=False),
        scratch_types=[
            pltpu.VMEM((_NEW_VOCAB, _EMBED_DIM), jnp.float32),
            pltpu.VMEM((2 * _STEP,), jnp.int32),
            pltpu.VMEM((2 * _STEP,), jnp.int32),
            pltpu.VMEM((2 * _STEP, _EMBED_DIM), jnp.float32),
            pltpu.VMEM((_EMBED_DIM, _STEP), jnp.float32),
            [pltpu.SemaphoreType.DMA] * 2,
            [pltpu.SemaphoreType.DMA] * 2,
            pltpu.SemaphoreType.DMA,
        ],
    )
    out_t = run(ids_t, old_table, new_table)
    return jnp.transpose(out_t, (2, 0, 1))
